# baseline (speedup 1.0000x reference)
import functools

import jax
import jax.numpy as jnp
from jax import lax
from jax.experimental import pallas as pl
from jax.experimental.pallas import tpu as pltpu
from jax.experimental.pallas import tpu_sc as plsc

N = 10000
E = 320000
D = 128
DH = D // 2
K = 3

NC = 2
NS = 16
NP = 10240
EP = 327680
CHUNK = 128
NCHUNKS = EP // CHUNK
GRP = 8
RPT = NP // NS
RCH = 128
NRC = RPT // RCH

NCT = NCHUNKS // NS
PG = 2
NGRP = NCT // PG
NSETS = 4

_MESH = plsc.VectorSubcoreMesh(core_axis_name="c", subcore_axis_name="s")


def _zero_sbuf(sbuf, width):
    def zrow(i, carry):
        for q in range(width // 16):
            sbuf[i, pl.ds(q * 16, 16)] = jnp.zeros((16,), jnp.float32)
        return carry
    lax.fori_loop(0, RCH, zrow, 0)


def _deg_body(rows_hbm, ones_hbm, deg0_hbm, deg1_hbm,
              idx8, onesv, sbuf, acc, sem):
    c = lax.axis_index("c")
    t = lax.axis_index("s")
    base = t * RPT

    _zero_sbuf(sbuf, 16)
    for k in range(NRC):
        pltpu.sync_copy(sbuf, acc.at[pl.ds(base + k * RCH, RCH)])
    pltpu.sync_copy(ones_hbm, onesv)
    plsc.subcore_barrier()

    ng = NCHUNKS // GRP
    gpt = ng // (NC * NS)
    g0 = (c * NS + t) * gpt

    def body(g, carry):
        pltpu.sync_copy(rows_hbm.at[pl.ds(g * GRP, GRP)], idx8)
        for r in range(GRP):
            pltpu.sync_copy(onesv, acc.at[idx8.at[r]], add=True)
        return carry
    lax.fori_loop(g0, g0 + gpt, body, 0)
    plsc.subcore_barrier()

    for k in range(NRC):
        s = pl.ds(base + k * RCH, RCH)
        pltpu.sync_copy(acc.at[s], sbuf)

        @pl.when(c == 0)
        def _():
            pltpu.sync_copy(sbuf, deg0_hbm.at[s])

        @pl.when(c == 1)
        def _():
            pltpu.sync_copy(sbuf, deg1_hbm.at[s])


_deg_call = pl.kernel(
    _deg_body,
    out_type=[jax.ShapeDtypeStruct((NP, 16), jnp.float32),
              jax.ShapeDtypeStruct((NP, 16), jnp.float32)],
    mesh=_MESH,
    compiler_params=pltpu.CompilerParams(use_tc_tiling_on_sc=False),
    scratch_types=[
        pltpu.VMEM((GRP, CHUNK), jnp.int32),
        pltpu.VMEM((CHUNK, 16), jnp.float32),
        pltpu.VMEM((RCH, 16), jnp.float32),
        pltpu.VMEM_SHARED((NP, 16), jnp.float32),
        pltpu.SemaphoreType.DMA,
    ],
)


def _pipelined_spmm(c, t, rows_hbm, cols_hbm, src, acc,
                    rbufs, cbufs, sets, gsems, ssems, isr, isc):
    g0 = t * NGRP

    def cs(g):
        return cols_hbm.at[c, pl.ds((g0 + g) * PG, PG)]

    def rs(g):
        return rows_hbm.at[pl.ds((g0 + g) * PG, PG)]

    for s in range(NSETS):
        pltpu.async_copy(cs(s), cbufs[s], isc[s])
    pltpu.async_copy(rs(0), rbufs[0], isr[0])
    pltpu.make_async_copy(cs(0), cbufs[0], isc[0]).wait()
    for r in range(PG):
        pltpu.async_copy(src.at[cbufs[0].at[r]], sets[0][r], gsems[0])

    def body(g, carry):
        for s in range(NSETS):
            nx = (s + 1) % NSETS

            @pl.when(lax.rem(g, NSETS) == s)
            def _(s=s, nx=nx):
                @pl.when(g >= NSETS - 1)
                def _():
                    for r in range(PG):
                        pltpu.make_async_copy(
                            sets[nx][r], acc.at[rbufs[nx].at[r]],
                            ssems[nx]).wait()

                @pl.when(g < NGRP - 1)
                def _():
                    pltpu.async_copy(rs(g + 1), rbufs[nx], isr[nx])
                    pltpu.make_async_copy(cs(g + 1), cbufs[nx],
                                          isc[nx]).wait()
                    for r in range(PG):
                        pltpu.async_copy(src.at[cbufs[nx].at[r]], sets[nx][r],
                                         gsems[nx])

                for r in range(PG):
                    pltpu.make_async_copy(src.at[cbufs[s].at[r]], sets[s][r],
                                          gsems[s]).wait()

                @pl.when(g + NSETS < NGRP)
                def _():
                    pltpu.async_copy(cs(g + NSETS), cbufs[s], isc[s])

                pltpu.make_async_copy(rs(g), rbufs[s], isr[s]).wait()
                for r in range(PG):
                    pltpu.async_copy(sets[s][r], acc.at[rbufs[s].at[r]],
                                     ssems[s], add=True)
        return carry
    lax.fori_loop(0, NGRP, body, 0)

    for g in range(NGRP - NSETS + 1, NGRP):
        sg = g % NSETS
        for r in range(PG):
            pltpu.make_async_copy(sets[sg][r], acc.at[rbufs[sg].at[r]],
                                  ssems[sg]).wait()


def _layer_body(rows_hbm, cols_hbm, s_hbm, c_hbm, w_hbm, u2_hbm,
                rb0, rb1, rb2, rb3, cb0, cb1, cb2, cb3,
                b0, b1, b2, b3, b4, b5, b6, b7, zbuf, acc,
                gs0, gs1, gs2, gs3, ss0, ss1, ss2, ss3,
                ir0, ir1, ir2, ir3, ic0, ic1, ic2, ic3):
    c = lax.axis_index("c")
    t = lax.axis_index("s")
    base = t * RPT
    rbufs = (rb0, rb1, rb2, rb3)
    cbufs = (cb0, cb1, cb2, cb3)
    sets = ((b0, b1), (b2, b3), (b4, b5), (b6, b7))
    gsems = (gs0, gs1, gs2, gs3)
    ssems = (ss0, ss1, ss2, ss3)
    isr = (ir0, ir1, ir2, ir3)
    isc = (ic0, ic1, ic2, ic3)
    sbuf, cv = b0, b1

    _zero_sbuf(zbuf, DH)
    for k in range(NRC):
        pltpu.sync_copy(zbuf, acc.at[pl.ds(base + k * RCH, RCH)])
    plsc.subcore_barrier()

    _pipelined_spmm(c, t, rows_hbm, cols_hbm, s_hbm, acc,
                    rbufs, cbufs, sets, gsems, ssems, isr, isc)
    plsc.subcore_barrier()

    obase = c * NP + base
    for k in range(NRC):
        sl = pl.ds(base + k * RCH, RCH)
        pltpu.sync_copy(acc.at[sl], sbuf)
        pltpu.sync_copy(zbuf, acc.at[sl])
        pltpu.sync_copy(c_hbm.at[sl], cv)

        def srow(i, carry):
            for q in range(DH // 16):
                ssl = (i, pl.ds(q * 16, 16))
                sbuf[ssl] = sbuf[ssl] * cv[ssl]
            return carry
        lax.fori_loop(0, RCH, srow, 0)
        pltpu.sync_copy(sbuf, w_hbm.at[pl.ds(obase + k * RCH, RCH)])
    plsc.subcore_barrier()

    _pipelined_spmm(c, t, rows_hbm, cols_hbm, w_hbm, acc,
                    rbufs, cbufs, sets, gsems, ssems, isr, isc)
    plsc.subcore_barrier()

    for k in range(NRC):
        pltpu.sync_copy(acc.at[pl.ds(base + k * RCH, RCH)], sbuf)
        pltpu.sync_copy(sbuf, u2_hbm.at[pl.ds(obase + k * RCH, RCH)])


_layer_call = pl.kernel(
    _layer_body,
    out_type=[jax.ShapeDtypeStruct((NC * NP, DH), jnp.float32),
              jax.ShapeDtypeStruct((NC * NP, DH), jnp.float32)],
    mesh=_MESH,
    compiler_params=pltpu.CompilerParams(use_tc_tiling_on_sc=False),
    scratch_types=(
        [pltpu.VMEM((PG, CHUNK), jnp.int32)] * 8
        + [pltpu.VMEM((CHUNK, DH), jnp.float32)] * 9
        + [pltpu.VMEM_SHARED((NP, DH), jnp.float32)]
        + [pltpu.SemaphoreType.DMA] * 16
    ),
)


BM = 640


def _prep_body(d0_ref, d1_ref, x_ref, dis_o, dsq_o, c_o, s_o):
    deg = d0_ref[:, 0:1] + d1_ref[:, 0:1]
    pos = deg > 0
    dis = jnp.where(pos, lax.rsqrt(jnp.maximum(deg, 1.0)), 0.0)
    dis_o[...] = dis
    dsq_o[...] = jnp.where(pos, jnp.sqrt(deg), 0.0)
    c_o[...] = jnp.broadcast_to(-(dis * dis), (BM, DH))
    s = x_ref[...] * dis
    s_o[...] = jnp.stack([s[:, :DH], s[:, DH:]])


def _prep(d0, d1, x):
    return pl.pallas_call(
        _prep_body,
        grid=(NP // BM,),
        in_specs=[
            pl.BlockSpec((BM, 16), lambda i: (i, 0)),
            pl.BlockSpec((BM, 16), lambda i: (i, 0)),
            pl.BlockSpec((BM, D), lambda i: (i, 0)),
        ],
        out_specs=[
            pl.BlockSpec((BM, 1), lambda i: (i, 0)),
            pl.BlockSpec((BM, 1), lambda i: (i, 0)),
            pl.BlockSpec((BM, DH), lambda i: (i, 0)),
            pl.BlockSpec((2, BM, DH), lambda i: (0, i, 0)),
        ],
        out_shape=[
            jax.ShapeDtypeStruct((NP, 1), jnp.float32),
            jax.ShapeDtypeStruct((NP, 1), jnp.float32),
            jax.ShapeDtypeStruct((NP, DH), jnp.float32),
            jax.ShapeDtypeStruct((2, NP, DH), jnp.float32),
        ],
    )(d0, d1, x)


def _combine_body(t_ref, w_ref, u2_ref, dis_ref, dsq_ref, wt_ref, b_ref,
                  *out_refs, relu):
    dis = dis_ref[...]
    tt = t_ref[...]
    wv = w_ref[...]
    u2v = u2_ref[...]
    tx1 = jnp.concatenate([wv[0], wv[1]], axis=1) * dsq_ref[...]
    tx2 = -2.0 * dis * jnp.concatenate([u2v[0], u2v[1]], axis=1) - tt
    wt = wt_ref[...]
    acc = jnp.dot(tt, wt[0], preferred_element_type=jnp.float32)
    acc = acc + jnp.dot(tx1, wt[1], preferred_element_type=jnp.float32)
    acc = acc + jnp.dot(tx2, wt[2], preferred_element_type=jnp.float32)
    acc = acc + b_ref[...]
    if relu:
        h = jnp.maximum(acc, 0.0)
        out_refs[0][...] = h
        s = h * dis
        out_refs[1][...] = jnp.stack([s[:, :DH], s[:, DH:]])
    else:
        out_refs[0][...] = acc


def _combine(t, w, u2, dis, dsq, wt, b, relu):
    if relu:
        out_shape = [
            jax.ShapeDtypeStruct((NP, D), jnp.float32),
            jax.ShapeDtypeStruct((2, NP, DH), jnp.float32),
        ]
        out_specs = [
            pl.BlockSpec((BM, D), lambda i: (i, 0)),
            pl.BlockSpec((2, BM, DH), lambda i: (0, i, 0)),
        ]
    else:
        out_shape = [jax.ShapeDtypeStruct((NP, D), jnp.float32)]
        out_specs = [pl.BlockSpec((BM, D), lambda i: (i, 0))]
    return pl.pallas_call(
        functools.partial(_combine_body, relu=relu),
        grid=(NP // BM,),
        in_specs=[
            pl.BlockSpec((BM, D), lambda i: (i, 0)),
            pl.BlockSpec((2, BM, DH), lambda i: (0, i, 0)),
            pl.BlockSpec((2, BM, DH), lambda i: (0, i, 0)),
            pl.BlockSpec((BM, 1), lambda i: (i, 0)),
            pl.BlockSpec((BM, 1), lambda i: (i, 0)),
            pl.BlockSpec((K, D, D), lambda i: (0, 0, 0)),
            pl.BlockSpec((1, D), lambda i: (0, 0)),
        ],
        out_specs=out_specs,
        out_shape=out_shape,
    )(t, w, u2, dis, dsq, wt, b)


def kernel(x, edge_index, W1, b1, W2, b2):
    ei = edge_index.astype(jnp.int32)
    pad = jnp.full((2, EP - E), N, jnp.int32)
    ei = jnp.concatenate([ei, pad], axis=1)
    rows = ei[0].reshape(NCHUNKS, CHUNK)
    cols = ei[1].reshape(NCHUNKS, CHUNK)
    cols2 = jnp.stack([cols, cols + NP])
    ones16 = jnp.ones((CHUNK, 16), jnp.float32)
    xp = jnp.pad(x, ((0, NP - N), (0, 0)))

    d0, d1 = _deg_call(rows, ones16)
    dis, dsq, cmat, s2 = _prep(d0, d1, xp)

    w1s, u2s = _layer_call(rows, cols2, s2.reshape(NC * NP, DH), cmat)
    h, s2b = _combine(xp, w1s.reshape(2, NP, DH), u2s.reshape(2, NP, DH),
                      dis, dsq, W1, b1.reshape(1, D), relu=True)
    w2s, v2s = _layer_call(rows, cols2, s2b.reshape(NC * NP, DH), cmat)
    (out,) = _combine(h, w2s.reshape(2, NP, DH), v2s.reshape(2, NP, DH),
                      dis, dsq, W2, b2.reshape(1, D), relu=False)
    return out[:N]


# speedup vs baseline: 1.0049x; 1.0049x over previous
import functools

import jax
import jax.numpy as jnp
from jax import lax
from jax.experimental import pallas as pl
from jax.experimental.pallas import tpu as pltpu
from jax.experimental.pallas import tpu_sc as plsc

N = 10000
E = 320000
D = 128
DH = D // 2
K = 3

NC = 2
NS = 16
NP = 10240
EP = 327680
CHUNK = 128
NCHUNKS = EP // CHUNK
GRP = 8
RPT = NP // NS
RCH = 128
NRC = RPT // RCH

NCT = NCHUNKS // NS
PG = 2
NGRP = NCT // PG
NSETS = 2

_MESH = plsc.VectorSubcoreMesh(core_axis_name="c", subcore_axis_name="s")


def _zero_sbuf(sbuf, width):
    def zrow(i, carry):
        for q in range(width // 16):
            sbuf[i, pl.ds(q * 16, 16)] = jnp.zeros((16,), jnp.float32)
        return carry
    lax.fori_loop(0, RCH, zrow, 0)


def _deg_body(rows_hbm, ones_hbm, deg0_hbm, deg1_hbm,
              idx8, onesv, sbuf, acc, sem):
    c = lax.axis_index("c")
    t = lax.axis_index("s")
    base = t * RPT

    _zero_sbuf(sbuf, 16)
    for k in range(NRC):
        pltpu.sync_copy(sbuf, acc.at[pl.ds(base + k * RCH, RCH)])
    pltpu.sync_copy(ones_hbm, onesv)
    plsc.subcore_barrier()

    ng = NCHUNKS // GRP
    gpt = ng // (NC * NS)
    g0 = (c * NS + t) * gpt

    def body(g, carry):
        pltpu.sync_copy(rows_hbm.at[pl.ds(g * GRP, GRP)], idx8)
        for r in range(GRP):
            pltpu.sync_copy(onesv, acc.at[idx8.at[r]], add=True)
        return carry
    lax.fori_loop(g0, g0 + gpt, body, 0)
    plsc.subcore_barrier()

    for k in range(NRC):
        s = pl.ds(base + k * RCH, RCH)
        pltpu.sync_copy(acc.at[s], sbuf)

        @pl.when(c == 0)
        def _():
            pltpu.sync_copy(sbuf, deg0_hbm.at[s])

        @pl.when(c == 1)
        def _():
            pltpu.sync_copy(sbuf, deg1_hbm.at[s])


_deg_call = pl.kernel(
    _deg_body,
    out_type=[jax.ShapeDtypeStruct((NP, 16), jnp.float32),
              jax.ShapeDtypeStruct((NP, 16), jnp.float32)],
    mesh=_MESH,
    compiler_params=pltpu.CompilerParams(use_tc_tiling_on_sc=False),
    scratch_types=[
        pltpu.VMEM((GRP, CHUNK), jnp.int32),
        pltpu.VMEM((CHUNK, 16), jnp.float32),
        pltpu.VMEM((RCH, 16), jnp.float32),
        pltpu.VMEM_SHARED((NP, 16), jnp.float32),
        pltpu.SemaphoreType.DMA,
    ],
)


def _pipelined_spmm(c, t, rows_hbm, cols_hbm, src, acc,
                    rbufs, cbufs, sets, gsems, ssems, isr, isc):
    g0 = t * NGRP

    def cs(g):
        return cols_hbm.at[c, pl.ds((g0 + g) * PG, PG)]

    def rs(g):
        return rows_hbm.at[pl.ds((g0 + g) * PG, PG)]

    for s in range(NSETS):
        pltpu.async_copy(cs(s), cbufs[s], isc[s])
    pltpu.async_copy(rs(0), rbufs[0], isr[0])
    pltpu.make_async_copy(cs(0), cbufs[0], isc[0]).wait()
    for r in range(PG):
        pltpu.async_copy(src.at[cbufs[0].at[r]], sets[0][r], gsems[0])

    def body(g, carry):
        for s in range(NSETS):
            nx = (s + 1) % NSETS

            @pl.when(lax.rem(g, NSETS) == s)
            def _(s=s, nx=nx):
                @pl.when(g >= NSETS - 1)
                def _():
                    for r in range(PG):
                        pltpu.make_async_copy(
                            sets[nx][r], acc.at[rbufs[nx].at[r]],
                            ssems[nx]).wait()

                @pl.when(g < NGRP - 1)
                def _():
                    pltpu.async_copy(rs(g + 1), rbufs[nx], isr[nx])
                    pltpu.make_async_copy(cs(g + 1), cbufs[nx],
                                          isc[nx]).wait()
                    for r in range(PG):
                        pltpu.async_copy(src.at[cbufs[nx].at[r]], sets[nx][r],
                                         gsems[nx])

                for r in range(PG):
                    pltpu.make_async_copy(src.at[cbufs[s].at[r]], sets[s][r],
                                          gsems[s]).wait()

                @pl.when(g + NSETS < NGRP)
                def _():
                    pltpu.async_copy(cs(g + NSETS), cbufs[s], isc[s])

                pltpu.make_async_copy(rs(g), rbufs[s], isr[s]).wait()
                for r in range(PG):
                    pltpu.async_copy(sets[s][r], acc.at[rbufs[s].at[r]],
                                     ssems[s], add=True)
        return carry
    lax.fori_loop(0, NGRP, body, 0)

    for g in range(NGRP - NSETS + 1, NGRP):
        sg = g % NSETS
        for r in range(PG):
            pltpu.make_async_copy(sets[sg][r], acc.at[rbufs[sg].at[r]],
                                  ssems[sg]).wait()


def _layer_body(rows_hbm, cols_hbm, s_hbm, c_hbm, w_hbm, u2_hbm,
                rb0, rb1, rb2, rb3, cb0, cb1, cb2, cb3,
                b0, b1, b2, b3, b4, b5, b6, b7, zbuf, acc,
                gs0, gs1, gs2, gs3, ss0, ss1, ss2, ss3,
                ir0, ir1, ir2, ir3, ic0, ic1, ic2, ic3):
    c = lax.axis_index("c")
    t = lax.axis_index("s")
    base = t * RPT
    rbufs = (rb0, rb1, rb2, rb3)[:NSETS]
    cbufs = (cb0, cb1, cb2, cb3)[:NSETS]
    sets = ((b0, b1), (b2, b3), (b4, b5), (b6, b7))[:NSETS]
    gsems = (gs0, gs1, gs2, gs3)[:NSETS]
    ssems = (ss0, ss1, ss2, ss3)[:NSETS]
    isr = (ir0, ir1, ir2, ir3)[:NSETS]
    isc = (ic0, ic1, ic2, ic3)[:NSETS]
    sbuf, cv = b0, b1

    _zero_sbuf(zbuf, DH)
    for k in range(NRC):
        pltpu.sync_copy(zbuf, acc.at[pl.ds(base + k * RCH, RCH)])
    plsc.subcore_barrier()

    _pipelined_spmm(c, t, rows_hbm, cols_hbm, s_hbm, acc,
                    rbufs, cbufs, sets, gsems, ssems, isr, isc)
    plsc.subcore_barrier()

    obase = c * NP + base
    for k in range(NRC):
        sl = pl.ds(base + k * RCH, RCH)
        pltpu.sync_copy(acc.at[sl], sbuf)
        pltpu.sync_copy(zbuf, acc.at[sl])
        pltpu.sync_copy(c_hbm.at[sl], cv)

        def srow(i, carry):
            for q in range(DH // 16):
                ssl = (i, pl.ds(q * 16, 16))
                sbuf[ssl] = sbuf[ssl] * cv[ssl]
            return carry
        lax.fori_loop(0, RCH, srow, 0)
        pltpu.sync_copy(sbuf, w_hbm.at[pl.ds(obase + k * RCH, RCH)])
    plsc.subcore_barrier()

    _pipelined_spmm(c, t, rows_hbm, cols_hbm, w_hbm, acc,
                    rbufs, cbufs, sets, gsems, ssems, isr, isc)
    plsc.subcore_barrier()

    for k in range(NRC):
        pltpu.sync_copy(acc.at[pl.ds(base + k * RCH, RCH)], sbuf)
        pltpu.sync_copy(sbuf, u2_hbm.at[pl.ds(obase + k * RCH, RCH)])


_layer_call = pl.kernel(
    _layer_body,
    out_type=[jax.ShapeDtypeStruct((NC * NP, DH), jnp.float32),
              jax.ShapeDtypeStruct((NC * NP, DH), jnp.float32)],
    mesh=_MESH,
    compiler_params=pltpu.CompilerParams(use_tc_tiling_on_sc=False),
    scratch_types=(
        [pltpu.VMEM((PG, CHUNK), jnp.int32)] * 8
        + [pltpu.VMEM((CHUNK, DH), jnp.float32)] * 9
        + [pltpu.VMEM_SHARED((NP, DH), jnp.float32)]
        + [pltpu.SemaphoreType.DMA] * 16
    ),
)


BM = 640


def _prep_body(d0_ref, d1_ref, x_ref, dis_o, dsq_o, c_o, s_o):
    deg = d0_ref[:, 0:1] + d1_ref[:, 0:1]
    pos = deg > 0
    dis = jnp.where(pos, lax.rsqrt(jnp.maximum(deg, 1.0)), 0.0)
    dis_o[...] = dis
    dsq_o[...] = jnp.where(pos, jnp.sqrt(deg), 0.0)
    c_o[...] = jnp.broadcast_to(-(dis * dis), (BM, DH))
    s = x_ref[...] * dis
    s_o[...] = jnp.stack([s[:, :DH], s[:, DH:]])


def _prep(d0, d1, x):
    return pl.pallas_call(
        _prep_body,
        grid=(NP // BM,),
        in_specs=[
            pl.BlockSpec((BM, 16), lambda i: (i, 0)),
            pl.BlockSpec((BM, 16), lambda i: (i, 0)),
            pl.BlockSpec((BM, D), lambda i: (i, 0)),
        ],
        out_specs=[
            pl.BlockSpec((BM, 1), lambda i: (i, 0)),
            pl.BlockSpec((BM, 1), lambda i: (i, 0)),
            pl.BlockSpec((BM, DH), lambda i: (i, 0)),
            pl.BlockSpec((2, BM, DH), lambda i: (0, i, 0)),
        ],
        out_shape=[
            jax.ShapeDtypeStruct((NP, 1), jnp.float32),
            jax.ShapeDtypeStruct((NP, 1), jnp.float32),
            jax.ShapeDtypeStruct((NP, DH), jnp.float32),
            jax.ShapeDtypeStruct((2, NP, DH), jnp.float32),
        ],
    )(d0, d1, x)


def _combine_body(t_ref, w_ref, u2_ref, dis_ref, dsq_ref, wt_ref, b_ref,
                  *out_refs, relu):
    dis = dis_ref[...]
    tt = t_ref[...]
    wv = w_ref[...]
    u2v = u2_ref[...]
    tx1 = jnp.concatenate([wv[0], wv[1]], axis=1) * dsq_ref[...]
    tx2 = -2.0 * dis * jnp.concatenate([u2v[0], u2v[1]], axis=1) - tt
    wt = wt_ref[...]
    acc = jnp.dot(tt, wt[0], preferred_element_type=jnp.float32)
    acc = acc + jnp.dot(tx1, wt[1], preferred_element_type=jnp.float32)
    acc = acc + jnp.dot(tx2, wt[2], preferred_element_type=jnp.float32)
    acc = acc + b_ref[...]
    if relu:
        h = jnp.maximum(acc, 0.0)
        out_refs[0][...] = h
        s = h * dis
        out_refs[1][...] = jnp.stack([s[:, :DH], s[:, DH:]])
    else:
        out_refs[0][...] = acc


def _combine(t, w, u2, dis, dsq, wt, b, relu):
    if relu:
        out_shape = [
            jax.ShapeDtypeStruct((NP, D), jnp.float32),
            jax.ShapeDtypeStruct((2, NP, DH), jnp.float32),
        ]
        out_specs = [
            pl.BlockSpec((BM, D), lambda i: (i, 0)),
            pl.BlockSpec((2, BM, DH), lambda i: (0, i, 0)),
        ]
    else:
        out_shape = [jax.ShapeDtypeStruct((NP, D), jnp.float32)]
        out_specs = [pl.BlockSpec((BM, D), lambda i: (i, 0))]
    return pl.pallas_call(
        functools.partial(_combine_body, relu=relu),
        grid=(NP // BM,),
        in_specs=[
            pl.BlockSpec((BM, D), lambda i: (i, 0)),
            pl.BlockSpec((2, BM, DH), lambda i: (0, i, 0)),
            pl.BlockSpec((2, BM, DH), lambda i: (0, i, 0)),
            pl.BlockSpec((BM, 1), lambda i: (i, 0)),
            pl.BlockSpec((BM, 1), lambda i: (i, 0)),
            pl.BlockSpec((K, D, D), lambda i: (0, 0, 0)),
            pl.BlockSpec((1, D), lambda i: (0, 0)),
        ],
        out_specs=out_specs,
        out_shape=out_shape,
    )(t, w, u2, dis, dsq, wt, b)


def kernel(x, edge_index, W1, b1, W2, b2):
    ei = edge_index.astype(jnp.int32)
    pad = jnp.full((2, EP - E), N, jnp.int32)
    ei = jnp.concatenate([ei, pad], axis=1)
    rows = ei[0].reshape(NCHUNKS, CHUNK)
    cols = ei[1].reshape(NCHUNKS, CHUNK)
    cols2 = jnp.stack([cols, cols + NP])
    ones16 = jnp.ones((CHUNK, 16), jnp.float32)
    xp = jnp.pad(x, ((0, NP - N), (0, 0)))

    d0, d1 = _deg_call(rows, ones16)
    dis, dsq, cmat, s2 = _prep(d0, d1, xp)

    w1s, u2s = _layer_call(rows, cols2, s2.reshape(NC * NP, DH), cmat)
    h, s2b = _combine(xp, w1s.reshape(2, NP, DH), u2s.reshape(2, NP, DH),
                      dis, dsq, W1, b1.reshape(1, D), relu=True)
    w2s, v2s = _layer_call(rows, cols2, s2b.reshape(NC * NP, DH), cmat)
    (out,) = _combine(h, w2s.reshape(2, NP, DH), v2s.reshape(2, NP, DH),
                      dis, dsq, W2, b2.reshape(1, D), relu=False)
    return out[:N]


# speedup vs baseline: 1.7120x; 1.7036x over previous
import functools

import jax
import jax.numpy as jnp
from jax import lax
from jax.experimental import pallas as pl
from jax.experimental.pallas import tpu as pltpu
from jax.experimental.pallas import tpu_sc as plsc

N = 10000
E = 320000
D = 128
DH = D // 2
K = 3

NC = 2
NS = 16
NP = 10240
EP = 327680
CHUNK = 128
NCHUNKS = EP // CHUNK
GRP = 8
RPT = NP // NS
RCH = 128
NRC = RPT // RCH

NCT = NCHUNKS // NS
PG = 2
NGRP = NCT // PG
NSETS = 2

_MESH = plsc.VectorSubcoreMesh(core_axis_name="c", subcore_axis_name="s")


def _zero_sbuf(sbuf, width):
    def zrow(i, carry):
        for q in range(width // 16):
            sbuf[i, pl.ds(q * 16, 16)] = jnp.zeros((16,), jnp.float32)
        return carry
    lax.fori_loop(0, RCH, zrow, 0)


def _deg_body(rows_hbm, ones_hbm, deg0_hbm, deg1_hbm,
              idx8, onesv, sbuf, acc, sem):
    c = lax.axis_index("c")
    t = lax.axis_index("s")
    base = t * RPT

    _zero_sbuf(sbuf, 16)
    for k in range(NRC):
        pltpu.sync_copy(sbuf, acc.at[pl.ds(base + k * RCH, RCH)])
    pltpu.sync_copy(ones_hbm, onesv)
    plsc.subcore_barrier()

    ng = NCHUNKS // GRP
    gpt = ng // (NC * NS)
    g0 = (c * NS + t) * gpt

    def body(g, carry):
        pltpu.sync_copy(rows_hbm.at[pl.ds(g * GRP, GRP)], idx8)
        for r in range(GRP):
            pltpu.sync_copy(onesv, acc.at[idx8.at[r]], add=True)
        return carry
    lax.fori_loop(g0, g0 + gpt, body, 0)
    plsc.subcore_barrier()

    for k in range(NRC):
        s = pl.ds(base + k * RCH, RCH)
        pltpu.sync_copy(acc.at[s], sbuf)

        @pl.when(c == 0)
        def _():
            pltpu.sync_copy(sbuf, deg0_hbm.at[s])

        @pl.when(c == 1)
        def _():
            pltpu.sync_copy(sbuf, deg1_hbm.at[s])


_deg_call = pl.kernel(
    _deg_body,
    out_type=[jax.ShapeDtypeStruct((NP, 16), jnp.float32),
              jax.ShapeDtypeStruct((NP, 16), jnp.float32)],
    mesh=_MESH,
    compiler_params=pltpu.CompilerParams(use_tc_tiling_on_sc=False),
    scratch_types=[
        pltpu.VMEM((GRP, CHUNK), jnp.int32),
        pltpu.VMEM((CHUNK, 16), jnp.float32),
        pltpu.VMEM((RCH, 16), jnp.float32),
        pltpu.VMEM_SHARED((NP, 16), jnp.float32),
        pltpu.SemaphoreType.DMA,
    ],
)


def _pipelined_spmm(c, t, rows_hbm, cols_hbm, src, acc,
                    rbufs, cbufs, sets, gsems, ssems, isr, isc):
    g0 = t * NGRP

    def cs(g):
        return cols_hbm.at[pl.ds((g0 + g) * PG, PG)]

    def rs(g):
        return rows_hbm.at[pl.ds((g0 + g) * PG, PG)]

    for s in range(NSETS):
        pltpu.async_copy(cs(s), cbufs[s], isc[s])
    pltpu.async_copy(rs(0), rbufs[0], isr[0])
    pltpu.make_async_copy(cs(0), cbufs[0], isc[0]).wait()
    for r in range(PG):
        pltpu.async_copy(src.at[cbufs[0].at[r]], sets[0][r], gsems[0])

    def body(g, carry):
        for s in range(NSETS):
            nx = (s + 1) % NSETS

            @pl.when(lax.rem(g, NSETS) == s)
            def _(s=s, nx=nx):
                @pl.when(g >= NSETS - 1)
                def _():
                    for r in range(PG):
                        pltpu.make_async_copy(
                            sets[nx][r], acc.at[rbufs[nx].at[r]],
                            ssems[nx]).wait()

                @pl.when(g < NGRP - 1)
                def _():
                    pltpu.async_copy(rs(g + 1), rbufs[nx], isr[nx])
                    pltpu.make_async_copy(cs(g + 1), cbufs[nx],
                                          isc[nx]).wait()
                    for r in range(PG):
                        pltpu.async_copy(src.at[cbufs[nx].at[r]], sets[nx][r],
                                         gsems[nx])

                for r in range(PG):
                    pltpu.make_async_copy(src.at[cbufs[s].at[r]], sets[s][r],
                                          gsems[s]).wait()

                @pl.when(g + NSETS < NGRP)
                def _():
                    pltpu.async_copy(cs(g + NSETS), cbufs[s], isc[s])

                pltpu.make_async_copy(rs(g), rbufs[s], isr[s]).wait()
                for r in range(PG):
                    pltpu.async_copy(sets[s][r], acc.at[rbufs[s].at[r]],
                                     ssems[s], add=True)
        return carry
    lax.fori_loop(0, NGRP, body, 0)

    for g in range(NGRP - NSETS + 1, NGRP):
        sg = g % NSETS
        for r in range(PG):
            pltpu.make_async_copy(sets[sg][r], acc.at[rbufs[sg].at[r]],
                                  ssems[sg]).wait()


def _layer_body(rows_hbm, cols_hbm, s_hbm, c_hbm, w_hbm, u2_hbm,
                rb0, rb1, rb2, rb3, cb0, cb1, cb2, cb3,
                b0, b1, b2, b3, b4, b5, b6, b7, zbuf, src, acc,
                gs0, gs1, gs2, gs3, ss0, ss1, ss2, ss3,
                ir0, ir1, ir2, ir3, ic0, ic1, ic2, ic3):
    c = lax.axis_index("c")
    t = lax.axis_index("s")
    base = t * RPT
    rbufs = (rb0, rb1, rb2, rb3)[:NSETS]
    cbufs = (cb0, cb1, cb2, cb3)[:NSETS]
    sets = ((b0, b1), (b2, b3), (b4, b5), (b6, b7))[:NSETS]
    gsems = (gs0, gs1, gs2, gs3)[:NSETS]
    ssems = (ss0, ss1, ss2, ss3)[:NSETS]
    isr = (ir0, ir1, ir2, ir3)[:NSETS]
    isc = (ic0, ic1, ic2, ic3)[:NSETS]
    sbuf, cv = b0, b1

    obase = c * NP + base
    _zero_sbuf(zbuf, DH)
    for k in range(NRC):
        sl = pl.ds(base + k * RCH, RCH)
        pltpu.sync_copy(zbuf, acc.at[sl])
        pltpu.sync_copy(s_hbm.at[pl.ds(obase + k * RCH, RCH)], sbuf)
        pltpu.sync_copy(sbuf, src.at[sl])
    plsc.subcore_barrier()

    _pipelined_spmm(c, t, rows_hbm, cols_hbm, src, acc,
                    rbufs, cbufs, sets, gsems, ssems, isr, isc)
    plsc.subcore_barrier()

    for k in range(NRC):
        sl = pl.ds(base + k * RCH, RCH)
        pltpu.sync_copy(acc.at[sl], sbuf)
        pltpu.sync_copy(zbuf, acc.at[sl])
        pltpu.sync_copy(c_hbm.at[sl], cv)

        def srow(i, carry):
            for q in range(DH // 16):
                ssl = (i, pl.ds(q * 16, 16))
                sbuf[ssl] = sbuf[ssl] * cv[ssl]
            return carry
        lax.fori_loop(0, RCH, srow, 0)
        pltpu.sync_copy(sbuf, w_hbm.at[pl.ds(obase + k * RCH, RCH)])
        pltpu.sync_copy(sbuf, src.at[sl])
    plsc.subcore_barrier()

    _pipelined_spmm(c, t, rows_hbm, cols_hbm, src, acc,
                    rbufs, cbufs, sets, gsems, ssems, isr, isc)
    plsc.subcore_barrier()

    for k in range(NRC):
        pltpu.sync_copy(acc.at[pl.ds(base + k * RCH, RCH)], sbuf)
        pltpu.sync_copy(sbuf, u2_hbm.at[pl.ds(obase + k * RCH, RCH)])


_layer_call = pl.kernel(
    _layer_body,
    out_type=[jax.ShapeDtypeStruct((NC * NP, DH), jnp.float32),
              jax.ShapeDtypeStruct((NC * NP, DH), jnp.float32)],
    mesh=_MESH,
    compiler_params=pltpu.CompilerParams(use_tc_tiling_on_sc=False),
    scratch_types=(
        [pltpu.VMEM((PG, CHUNK), jnp.int32)] * 8
        + [pltpu.VMEM((CHUNK, DH), jnp.float32)] * 9
        + [pltpu.VMEM_SHARED((NP, DH), jnp.float32)] * 2
        + [pltpu.SemaphoreType.DMA] * 16
    ),
)


BM = 640


def _prep_body(d0_ref, d1_ref, x_ref, dis_o, dsq_o, c_o, s_o):
    deg = d0_ref[:, 0:1] + d1_ref[:, 0:1]
    pos = deg > 0
    dis = jnp.where(pos, lax.rsqrt(jnp.maximum(deg, 1.0)), 0.0)
    dis_o[...] = dis
    dsq_o[...] = jnp.where(pos, jnp.sqrt(deg), 0.0)
    c_o[...] = jnp.broadcast_to(-(dis * dis), (BM, DH))
    s = x_ref[...] * dis
    s_o[...] = jnp.stack([s[:, :DH], s[:, DH:]])


def _prep(d0, d1, x):
    return pl.pallas_call(
        _prep_body,
        grid=(NP // BM,),
        in_specs=[
            pl.BlockSpec((BM, 16), lambda i: (i, 0)),
            pl.BlockSpec((BM, 16), lambda i: (i, 0)),
            pl.BlockSpec((BM, D), lambda i: (i, 0)),
        ],
        out_specs=[
            pl.BlockSpec((BM, 1), lambda i: (i, 0)),
            pl.BlockSpec((BM, 1), lambda i: (i, 0)),
            pl.BlockSpec((BM, DH), lambda i: (i, 0)),
            pl.BlockSpec((2, BM, DH), lambda i: (0, i, 0)),
        ],
        out_shape=[
            jax.ShapeDtypeStruct((NP, 1), jnp.float32),
            jax.ShapeDtypeStruct((NP, 1), jnp.float32),
            jax.ShapeDtypeStruct((NP, DH), jnp.float32),
            jax.ShapeDtypeStruct((2, NP, DH), jnp.float32),
        ],
    )(d0, d1, x)


def _combine_body(t_ref, w_ref, u2_ref, dis_ref, dsq_ref, wt_ref, b_ref,
                  *out_refs, relu):
    dis = dis_ref[...]
    tt = t_ref[...]
    wv = w_ref[...]
    u2v = u2_ref[...]
    tx1 = jnp.concatenate([wv[0], wv[1]], axis=1) * dsq_ref[...]
    tx2 = -2.0 * dis * jnp.concatenate([u2v[0], u2v[1]], axis=1) - tt
    wt = wt_ref[...]
    acc = jnp.dot(tt, wt[0], preferred_element_type=jnp.float32)
    acc = acc + jnp.dot(tx1, wt[1], preferred_element_type=jnp.float32)
    acc = acc + jnp.dot(tx2, wt[2], preferred_element_type=jnp.float32)
    acc = acc + b_ref[...]
    if relu:
        h = jnp.maximum(acc, 0.0)
        out_refs[0][...] = h
        s = h * dis
        out_refs[1][...] = jnp.stack([s[:, :DH], s[:, DH:]])
    else:
        out_refs[0][...] = acc


def _combine(t, w, u2, dis, dsq, wt, b, relu):
    if relu:
        out_shape = [
            jax.ShapeDtypeStruct((NP, D), jnp.float32),
            jax.ShapeDtypeStruct((2, NP, DH), jnp.float32),
        ]
        out_specs = [
            pl.BlockSpec((BM, D), lambda i: (i, 0)),
            pl.BlockSpec((2, BM, DH), lambda i: (0, i, 0)),
        ]
    else:
        out_shape = [jax.ShapeDtypeStruct((NP, D), jnp.float32)]
        out_specs = [pl.BlockSpec((BM, D), lambda i: (i, 0))]
    return pl.pallas_call(
        functools.partial(_combine_body, relu=relu),
        grid=(NP // BM,),
        in_specs=[
            pl.BlockSpec((BM, D), lambda i: (i, 0)),
            pl.BlockSpec((2, BM, DH), lambda i: (0, i, 0)),
            pl.BlockSpec((2, BM, DH), lambda i: (0, i, 0)),
            pl.BlockSpec((BM, 1), lambda i: (i, 0)),
            pl.BlockSpec((BM, 1), lambda i: (i, 0)),
            pl.BlockSpec((K, D, D), lambda i: (0, 0, 0)),
            pl.BlockSpec((1, D), lambda i: (0, 0)),
        ],
        out_specs=out_specs,
        out_shape=out_shape,
    )(t, w, u2, dis, dsq, wt, b)


def kernel(x, edge_index, W1, b1, W2, b2):
    ei = edge_index.astype(jnp.int32)
    pad = jnp.full((2, EP - E), N, jnp.int32)
    ei = jnp.concatenate([ei, pad], axis=1)
    rows = ei[0].reshape(NCHUNKS, CHUNK)
    cols = ei[1].reshape(NCHUNKS, CHUNK)
    ones16 = jnp.ones((CHUNK, 16), jnp.float32)
    xp = jnp.pad(x, ((0, NP - N), (0, 0)))

    d0, d1 = _deg_call(rows, ones16)
    dis, dsq, cmat, s2 = _prep(d0, d1, xp)

    w1s, u2s = _layer_call(rows, cols, s2.reshape(NC * NP, DH), cmat)
    h, s2b = _combine(xp, w1s.reshape(2, NP, DH), u2s.reshape(2, NP, DH),
                      dis, dsq, W1, b1.reshape(1, D), relu=True)
    w2s, v2s = _layer_call(rows, cols, s2b.reshape(NC * NP, DH), cmat)
    (out,) = _combine(h, w2s.reshape(2, NP, DH), v2s.reshape(2, NP, DH),
                      dis, dsq, W2, b2.reshape(1, D), relu=False)
    return out[:N]


# speedup vs baseline: 1.8971x; 1.1081x over previous
import functools

import jax
import jax.numpy as jnp
from jax import lax
from jax.experimental import pallas as pl
from jax.experimental.pallas import tpu as pltpu
from jax.experimental.pallas import tpu_sc as plsc

N = 10000
E = 320000
D = 128
DH = D // 2
K = 3

NC = 2
NS = 16
NP = 10240
EP = 327680
CHUNK = 64
NCHUNKS = EP // CHUNK
GRP = 8
RPT = NP // NS
RCH = 64
NRC = RPT // RCH

NCT = NCHUNKS // NS
PG = 2
NGRP = NCT // PG
NSETS = 4

_MESH = plsc.VectorSubcoreMesh(core_axis_name="c", subcore_axis_name="s")


def _zero_sbuf(sbuf, width):
    def zrow(i, carry):
        for q in range(width // 16):
            sbuf[i, pl.ds(q * 16, 16)] = jnp.zeros((16,), jnp.float32)
        return carry
    lax.fori_loop(0, RCH, zrow, 0)


def _deg_body(rows_hbm, ones_hbm, deg0_hbm, deg1_hbm,
              idx8, onesv, sbuf, acc, sem):
    c = lax.axis_index("c")
    t = lax.axis_index("s")
    base = t * RPT

    _zero_sbuf(sbuf, 16)
    for k in range(NRC):
        pltpu.sync_copy(sbuf, acc.at[pl.ds(base + k * RCH, RCH)])
    pltpu.sync_copy(ones_hbm, onesv)
    plsc.subcore_barrier()

    ng = NCHUNKS // GRP
    gpt = ng // (NC * NS)
    g0 = (c * NS + t) * gpt

    def body(g, carry):
        pltpu.sync_copy(rows_hbm.at[pl.ds(g * GRP, GRP)], idx8)
        for r in range(GRP):
            pltpu.sync_copy(onesv, acc.at[idx8.at[r]], add=True)
        return carry
    lax.fori_loop(g0, g0 + gpt, body, 0)
    plsc.subcore_barrier()

    for k in range(NRC):
        s = pl.ds(base + k * RCH, RCH)
        pltpu.sync_copy(acc.at[s], sbuf)

        @pl.when(c == 0)
        def _():
            pltpu.sync_copy(sbuf, deg0_hbm.at[s])

        @pl.when(c == 1)
        def _():
            pltpu.sync_copy(sbuf, deg1_hbm.at[s])


_deg_call = pl.kernel(
    _deg_body,
    out_type=[jax.ShapeDtypeStruct((NP, 16), jnp.float32),
              jax.ShapeDtypeStruct((NP, 16), jnp.float32)],
    mesh=_MESH,
    compiler_params=pltpu.CompilerParams(use_tc_tiling_on_sc=False),
    scratch_types=[
        pltpu.VMEM((GRP, CHUNK), jnp.int32),
        pltpu.VMEM((CHUNK, 16), jnp.float32),
        pltpu.VMEM((RCH, 16), jnp.float32),
        pltpu.VMEM_SHARED((NP, 16), jnp.float32),
        pltpu.SemaphoreType.DMA,
    ],
)


def _pipelined_spmm(c, t, rows_hbm, cols_hbm, src, acc,
                    rbufs, cbufs, sets, gsems, ssems, isr, isc):
    g0 = t * NGRP

    def cs(g):
        return cols_hbm.at[pl.ds((g0 + g) * PG, PG)]

    def rs(g):
        return rows_hbm.at[pl.ds((g0 + g) * PG, PG)]

    for s in range(NSETS):
        pltpu.async_copy(cs(s), cbufs[s], isc[s])
    pltpu.async_copy(rs(0), rbufs[0], isr[0])
    pltpu.make_async_copy(cs(0), cbufs[0], isc[0]).wait()
    for r in range(PG):
        pltpu.async_copy(src.at[cbufs[0].at[r]], sets[0][r], gsems[0])

    def body(g, carry):
        for s in range(NSETS):
            nx = (s + 1) % NSETS

            @pl.when(lax.rem(g, NSETS) == s)
            def _(s=s, nx=nx):
                @pl.when(g >= NSETS - 1)
                def _():
                    for r in range(PG):
                        pltpu.make_async_copy(
                            sets[nx][r], acc.at[rbufs[nx].at[r]],
                            ssems[nx]).wait()

                @pl.when(g < NGRP - 1)
                def _():
                    pltpu.async_copy(rs(g + 1), rbufs[nx], isr[nx])
                    pltpu.make_async_copy(cs(g + 1), cbufs[nx],
                                          isc[nx]).wait()
                    for r in range(PG):
                        pltpu.async_copy(src.at[cbufs[nx].at[r]], sets[nx][r],
                                         gsems[nx])

                for r in range(PG):
                    pltpu.make_async_copy(src.at[cbufs[s].at[r]], sets[s][r],
                                          gsems[s]).wait()

                @pl.when(g + NSETS < NGRP)
                def _():
                    pltpu.async_copy(cs(g + NSETS), cbufs[s], isc[s])

                pltpu.make_async_copy(rs(g), rbufs[s], isr[s]).wait()
                for r in range(PG):
                    pltpu.async_copy(sets[s][r], acc.at[rbufs[s].at[r]],
                                     ssems[s], add=True)
        return carry
    lax.fori_loop(0, NGRP, body, 0)

    for g in range(NGRP - NSETS + 1, NGRP):
        sg = g % NSETS
        for r in range(PG):
            pltpu.make_async_copy(sets[sg][r], acc.at[rbufs[sg].at[r]],
                                  ssems[sg]).wait()


def _layer_body(rows_hbm, cols_hbm, s_hbm, c_hbm, w_hbm, u2_hbm,
                rb0, rb1, rb2, rb3, cb0, cb1, cb2, cb3,
                b0, b1, b2, b3, b4, b5, b6, b7, zbuf, src, acc,
                gs0, gs1, gs2, gs3, ss0, ss1, ss2, ss3,
                ir0, ir1, ir2, ir3, ic0, ic1, ic2, ic3):
    c = lax.axis_index("c")
    t = lax.axis_index("s")
    base = t * RPT
    rbufs = (rb0, rb1, rb2, rb3)[:NSETS]
    cbufs = (cb0, cb1, cb2, cb3)[:NSETS]
    sets = ((b0, b1), (b2, b3), (b4, b5), (b6, b7))[:NSETS]
    gsems = (gs0, gs1, gs2, gs3)[:NSETS]
    ssems = (ss0, ss1, ss2, ss3)[:NSETS]
    isr = (ir0, ir1, ir2, ir3)[:NSETS]
    isc = (ic0, ic1, ic2, ic3)[:NSETS]
    sbuf, cv = b0, b1

    obase = c * NP + base
    _zero_sbuf(zbuf, DH)
    for k in range(NRC):
        sl = pl.ds(base + k * RCH, RCH)
        pltpu.sync_copy(zbuf, acc.at[sl])
        pltpu.sync_copy(s_hbm.at[pl.ds(obase + k * RCH, RCH)], sbuf)
        pltpu.sync_copy(sbuf, src.at[sl])
    plsc.subcore_barrier()

    _pipelined_spmm(c, t, rows_hbm, cols_hbm, src, acc,
                    rbufs, cbufs, sets, gsems, ssems, isr, isc)
    plsc.subcore_barrier()

    for k in range(NRC):
        sl = pl.ds(base + k * RCH, RCH)
        pltpu.sync_copy(acc.at[sl], sbuf)
        pltpu.sync_copy(zbuf, acc.at[sl])
        pltpu.sync_copy(c_hbm.at[sl], cv)

        def srow(i, carry):
            for q in range(DH // 16):
                ssl = (i, pl.ds(q * 16, 16))
                sbuf[ssl] = sbuf[ssl] * cv[ssl]
            return carry
        lax.fori_loop(0, RCH, srow, 0)
        pltpu.sync_copy(sbuf, w_hbm.at[pl.ds(obase + k * RCH, RCH)])
        pltpu.sync_copy(sbuf, src.at[sl])
    plsc.subcore_barrier()

    _pipelined_spmm(c, t, rows_hbm, cols_hbm, src, acc,
                    rbufs, cbufs, sets, gsems, ssems, isr, isc)
    plsc.subcore_barrier()

    for k in range(NRC):
        pltpu.sync_copy(acc.at[pl.ds(base + k * RCH, RCH)], sbuf)
        pltpu.sync_copy(sbuf, u2_hbm.at[pl.ds(obase + k * RCH, RCH)])


_layer_call = pl.kernel(
    _layer_body,
    out_type=[jax.ShapeDtypeStruct((NC * NP, DH), jnp.float32),
              jax.ShapeDtypeStruct((NC * NP, DH), jnp.float32)],
    mesh=_MESH,
    compiler_params=pltpu.CompilerParams(use_tc_tiling_on_sc=False),
    scratch_types=(
        [pltpu.VMEM((PG, CHUNK), jnp.int32)] * 8
        + [pltpu.VMEM((CHUNK, DH), jnp.float32)] * 9
        + [pltpu.VMEM_SHARED((NP, DH), jnp.float32)] * 2
        + [pltpu.SemaphoreType.DMA] * 16
    ),
)


BM = 640


def _prep_body(d0_ref, d1_ref, x_ref, dis_o, dsq_o, c_o, s_o):
    deg = d0_ref[:, 0:1] + d1_ref[:, 0:1]
    pos = deg > 0
    dis = jnp.where(pos, lax.rsqrt(jnp.maximum(deg, 1.0)), 0.0)
    dis_o[...] = dis
    dsq_o[...] = jnp.where(pos, jnp.sqrt(deg), 0.0)
    c_o[...] = jnp.broadcast_to(-(dis * dis), (BM, DH))
    s = x_ref[...] * dis
    s_o[...] = jnp.stack([s[:, :DH], s[:, DH:]])


def _prep(d0, d1, x):
    return pl.pallas_call(
        _prep_body,
        grid=(NP // BM,),
        in_specs=[
            pl.BlockSpec((BM, 16), lambda i: (i, 0)),
            pl.BlockSpec((BM, 16), lambda i: (i, 0)),
            pl.BlockSpec((BM, D), lambda i: (i, 0)),
        ],
        out_specs=[
            pl.BlockSpec((BM, 1), lambda i: (i, 0)),
            pl.BlockSpec((BM, 1), lambda i: (i, 0)),
            pl.BlockSpec((BM, DH), lambda i: (i, 0)),
            pl.BlockSpec((2, BM, DH), lambda i: (0, i, 0)),
        ],
        out_shape=[
            jax.ShapeDtypeStruct((NP, 1), jnp.float32),
            jax.ShapeDtypeStruct((NP, 1), jnp.float32),
            jax.ShapeDtypeStruct((NP, DH), jnp.float32),
            jax.ShapeDtypeStruct((2, NP, DH), jnp.float32),
        ],
    )(d0, d1, x)


def _combine_body(t_ref, w_ref, u2_ref, dis_ref, dsq_ref, wt_ref, b_ref,
                  *out_refs, relu):
    dis = dis_ref[...]
    tt = t_ref[...]
    wv = w_ref[...]
    u2v = u2_ref[...]
    tx1 = jnp.concatenate([wv[0], wv[1]], axis=1) * dsq_ref[...]
    tx2 = -2.0 * dis * jnp.concatenate([u2v[0], u2v[1]], axis=1) - tt
    wt = wt_ref[...]
    acc = jnp.dot(tt, wt[0], preferred_element_type=jnp.float32)
    acc = acc + jnp.dot(tx1, wt[1], preferred_element_type=jnp.float32)
    acc = acc + jnp.dot(tx2, wt[2], preferred_element_type=jnp.float32)
    acc = acc + b_ref[...]
    if relu:
        h = jnp.maximum(acc, 0.0)
        out_refs[0][...] = h
        s = h * dis
        out_refs[1][...] = jnp.stack([s[:, :DH], s[:, DH:]])
    else:
        out_refs[0][...] = acc


def _combine(t, w, u2, dis, dsq, wt, b, relu):
    if relu:
        out_shape = [
            jax.ShapeDtypeStruct((NP, D), jnp.float32),
            jax.ShapeDtypeStruct((2, NP, DH), jnp.float32),
        ]
        out_specs = [
            pl.BlockSpec((BM, D), lambda i: (i, 0)),
            pl.BlockSpec((2, BM, DH), lambda i: (0, i, 0)),
        ]
    else:
        out_shape = [jax.ShapeDtypeStruct((NP, D), jnp.float32)]
        out_specs = [pl.BlockSpec((BM, D), lambda i: (i, 0))]
    return pl.pallas_call(
        functools.partial(_combine_body, relu=relu),
        grid=(NP // BM,),
        in_specs=[
            pl.BlockSpec((BM, D), lambda i: (i, 0)),
            pl.BlockSpec((2, BM, DH), lambda i: (0, i, 0)),
            pl.BlockSpec((2, BM, DH), lambda i: (0, i, 0)),
            pl.BlockSpec((BM, 1), lambda i: (i, 0)),
            pl.BlockSpec((BM, 1), lambda i: (i, 0)),
            pl.BlockSpec((K, D, D), lambda i: (0, 0, 0)),
            pl.BlockSpec((1, D), lambda i: (0, 0)),
        ],
        out_specs=out_specs,
        out_shape=out_shape,
    )(t, w, u2, dis, dsq, wt, b)


def kernel(x, edge_index, W1, b1, W2, b2):
    ei = edge_index.astype(jnp.int32)
    pad = jnp.full((2, EP - E), N, jnp.int32)
    ei = jnp.concatenate([ei, pad], axis=1)
    rows = ei[0].reshape(NCHUNKS, CHUNK)
    cols = ei[1].reshape(NCHUNKS, CHUNK)
    ones16 = jnp.ones((CHUNK, 16), jnp.float32)
    xp = jnp.pad(x, ((0, NP - N), (0, 0)))

    d0, d1 = _deg_call(rows, ones16)
    dis, dsq, cmat, s2 = _prep(d0, d1, xp)

    w1s, u2s = _layer_call(rows, cols, s2.reshape(NC * NP, DH), cmat)
    h, s2b = _combine(xp, w1s.reshape(2, NP, DH), u2s.reshape(2, NP, DH),
                      dis, dsq, W1, b1.reshape(1, D), relu=True)
    w2s, v2s = _layer_call(rows, cols, s2b.reshape(NC * NP, DH), cmat)
    (out,) = _combine(h, w2s.reshape(2, NP, DH), v2s.reshape(2, NP, DH),
                      dis, dsq, W2, b2.reshape(1, D), relu=False)
    return out[:N]
